# Initial kernel scaffold; baseline (speedup 1.0000x reference)
import jax
import jax.numpy as jnp
from jax.experimental import pallas as pl


def kernel(x):
    raise NotImplementedError("write your pallas kernel here")


# speedup vs baseline: 3.6529x; 3.6529x over previous
import functools

import jax
import jax.numpy as jnp
from jax.experimental import pallas as pl

_N_ROWS = 131072
_N_COLS = 64
_BLOCK_ROWS = 8192


def _act_kernel(x_ref, o_ref):
    x = x_ref[...]
    col = jax.lax.broadcasted_iota(jnp.int32, x.shape, 1) % 9
    ax = jnp.abs(x)

    c1 = jnp.where((col == 2) | (col == 8), -1.0,
                   jnp.where(col == 3, -2.0, 0.0)).astype(jnp.float32)
    c2 = jnp.where(col == 6, -0.5, 0.0).astype(jnp.float32)
    e = jnp.exp(c1 * ax + c2 * x * x)

    l1p = jnp.log1p(e)
    s = jnp.sin(x + jnp.where(col == 5, jnp.float32(jnp.pi / 2), 0.0))

    relu = jnp.maximum(x, 0.0)
    den = 1.0 + e
    sig = jnp.where(x >= 0, 1.0, e) / den
    th = jnp.sign(x) * (1.0 - e) / den
    sp = relu + l1p

    out = jnp.where(col == 0, x,
          jnp.where(col == 1, relu,
          jnp.where(col == 2, sig,
          jnp.where(col == 3, th,
          jnp.where(col <= 5, s,
          jnp.where(col == 6, e,
          jnp.where(col == 7, ax, sp)))))))
    o_ref[...] = out


@jax.jit
def kernel(x):
    grid = (_N_ROWS // _BLOCK_ROWS,)
    return pl.pallas_call(
        _act_kernel,
        grid=grid,
        in_specs=[pl.BlockSpec((_BLOCK_ROWS, _N_COLS), lambda i: (i, 0))],
        out_specs=pl.BlockSpec((_BLOCK_ROWS, _N_COLS), lambda i: (i, 0)),
        out_shape=jax.ShapeDtypeStruct((_N_ROWS, _N_COLS), jnp.float32),
    )(x)


# speedup vs baseline: 4.3985x; 1.2041x over previous
import functools

import jax
import jax.numpy as jnp
from jax.experimental import pallas as pl

_N_ROWS = 65536
_N_COLS = 128
_BLOCK_ROWS = 4096


def _act_kernel(x_ref, o_ref):
    x = x_ref[...]
    col = jax.lax.broadcasted_iota(jnp.int32, x.shape, 1) % 64 % 9
    ax = jnp.abs(x)

    c1 = jnp.where((col == 2) | (col == 8), -1.0,
                   jnp.where(col == 3, -2.0, 0.0)).astype(jnp.float32)
    c2 = jnp.where(col == 6, -0.5, 0.0).astype(jnp.float32)
    e = jnp.exp(c1 * ax + c2 * x * x)

    l1p = jnp.log1p(e)
    s = jnp.sin(x + jnp.where(col == 5, jnp.float32(jnp.pi / 2), 0.0))

    relu = jnp.maximum(x, 0.0)
    den = 1.0 + e
    sig = jnp.where(x >= 0, 1.0, e) / den
    th = jnp.sign(x) * (1.0 - e) / den
    sp = relu + l1p

    out = jnp.where(col == 0, x,
          jnp.where(col == 1, relu,
          jnp.where(col == 2, sig,
          jnp.where(col == 3, th,
          jnp.where(col <= 5, s,
          jnp.where(col == 6, e,
          jnp.where(col == 7, ax, sp)))))))
    o_ref[...] = out


@jax.jit
def kernel(x):
    rows, cols = x.shape
    xr = x.reshape(_N_ROWS, _N_COLS)
    grid = (_N_ROWS // _BLOCK_ROWS,)
    out = pl.pallas_call(
        _act_kernel,
        grid=grid,
        in_specs=[pl.BlockSpec((_BLOCK_ROWS, _N_COLS), lambda i: (i, 0))],
        out_specs=pl.BlockSpec((_BLOCK_ROWS, _N_COLS), lambda i: (i, 0)),
        out_shape=jax.ShapeDtypeStruct((_N_ROWS, _N_COLS), jnp.float32),
    )(xr)
    return out.reshape(rows, cols)


# speedup vs baseline: 6.3274x; 1.4385x over previous
import jax
import jax.numpy as jnp
from jax.experimental import pallas as pl

_N_ROWS = 65536
_N_COLS = 128
_BLOCK_ROWS = 4096

_PI_HI = 3.140625
_PI_LO = 9.67653589793e-4
_INV_PI = 0.3183098861837907
_HALF_PI = 1.5707963267948966

_S3 = -1.6666667163e-01
_S5 = 8.3333337680e-03
_S7 = -1.9841270114e-04
_S9 = 2.7557314297e-06


def _sin_poly(t):
    n = jnp.round(t * jnp.float32(_INV_PI))
    odd = jax.lax.shift_left(n.astype(jnp.int32), 31)
    r = t - n * jnp.float32(_PI_HI)
    r = r - n * jnp.float32(_PI_LO)
    r2 = r * r
    p = r + r * (r2 * (_S3 + r2 * (_S5 + r2 * (_S7 + r2 * _S9))))
    return (p.view(jnp.int32) ^ odd).view(jnp.float32)


def _act_kernel(x_ref, o_ref):
    x = x_ref[...]
    col = jax.lax.broadcasted_iota(jnp.int32, (1, _N_COLS), 1) % 64 % 9

    ax = jnp.abs(x)
    relu = jnp.maximum(x, 0.0)

    earg = jnp.where(col == 6, x * x * -0.5, -ax)
    e = jnp.exp(earg)
    den = 1.0 + e
    rden = 1.0 / den
    sig = jnp.where(x >= 0, 1.0, e) * rden
    sp = relu + jnp.log(den)
    th = jnp.tanh(x)
    s = _sin_poly(x + jnp.where(col == 5, jnp.float32(_HALF_PI), 0.0))

    out = jnp.where(col == 0, x,
          jnp.where(col == 1, relu,
          jnp.where(col == 2, sig,
          jnp.where(col == 3, th,
          jnp.where(col <= 5, s,
          jnp.where(col == 6, e,
          jnp.where(col == 7, ax, sp)))))))
    o_ref[...] = out


@jax.jit
def kernel(x):
    rows, cols = x.shape
    xr = x.reshape(_N_ROWS, _N_COLS)
    grid = (_N_ROWS // _BLOCK_ROWS,)
    out = pl.pallas_call(
        _act_kernel,
        grid=grid,
        in_specs=[pl.BlockSpec((_BLOCK_ROWS, _N_COLS), lambda i: (i, 0))],
        out_specs=pl.BlockSpec((_BLOCK_ROWS, _N_COLS), lambda i: (i, 0)),
        out_shape=jax.ShapeDtypeStruct((_N_ROWS, _N_COLS), jnp.float32),
    )(xr)
    return out.reshape(rows, cols)


# speedup vs baseline: 7.5017x; 1.1856x over previous
import jax
import jax.numpy as jnp
from jax.experimental import pallas as pl

_N_ROWS = 131072
_N_COLS = 64
_BLOCK_ROWS = 8192

_PI_HI = 3.140625
_PI_LO = 9.67653589793e-4
_INV_PI = 0.3183098861837907
_HALF_PI = 1.5707963267948966

_S3 = -1.6666667163e-01
_S5 = 8.3333337680e-03
_S7 = -1.9841270114e-04
_S9 = 2.7557314297e-06


def _sin_poly(t):
    n = jnp.round(t * jnp.float32(_INV_PI))
    odd = jax.lax.shift_left(n.astype(jnp.int32), 31)
    r = t - n * jnp.float32(_PI_HI)
    r = r - n * jnp.float32(_PI_LO)
    r2 = r * r
    p = r + r * (r2 * (_S3 + r2 * (_S5 + r2 * (_S7 + r2 * _S9))))
    return (p.view(jnp.int32) ^ odd).view(jnp.float32)


def _act_kernel(x_ref, o_ref):
    x = x_ref[...]
    col = jax.lax.broadcasted_iota(jnp.int32, (1, _N_COLS), 1) % 9

    ax = jnp.abs(x)
    relu = jnp.maximum(x, 0.0)

    earg = jnp.where(col == 6, x * x * -0.5, -ax)
    e = jnp.exp(earg)
    den = 1.0 + e
    rden = 1.0 / den
    sig = jnp.where(x >= 0, 1.0, e) * rden
    sp = relu + jnp.log(den)
    th = jnp.tanh(x)
    s = _sin_poly(x + jnp.where(col == 5, jnp.float32(_HALF_PI), 0.0))

    out = jnp.where(col == 0, x,
          jnp.where(col == 1, relu,
          jnp.where(col == 2, sig,
          jnp.where(col == 3, th,
          jnp.where(col <= 5, s,
          jnp.where(col == 6, e,
          jnp.where(col == 7, ax, sp)))))))
    o_ref[...] = out


@jax.jit
def kernel(x):
    grid = (_N_ROWS // _BLOCK_ROWS,)
    return pl.pallas_call(
        _act_kernel,
        grid=grid,
        in_specs=[pl.BlockSpec((_BLOCK_ROWS, _N_COLS), lambda i: (i, 0))],
        out_specs=pl.BlockSpec((_BLOCK_ROWS, _N_COLS), lambda i: (i, 0)),
        out_shape=jax.ShapeDtypeStruct((_N_ROWS, _N_COLS), jnp.float32),
    )(x)


# speedup vs baseline: 7.5342x; 1.0043x over previous
import jax
import jax.numpy as jnp
from jax.experimental import pallas as pl

_N_ROWS = 131072
_N_COLS = 64
_BLOCK_ROWS = 4096

_PI_HI = 3.140625
_PI_LO = 9.67653589793e-4
_INV_PI = 0.3183098861837907
_HALF_PI = 1.5707963267948966

_S3 = -1.6666667163e-01
_S5 = 8.3333337680e-03
_S7 = -1.9841270114e-04
_S9 = 2.7557314297e-06


def _sin_poly(t):
    n = jnp.round(t * jnp.float32(_INV_PI))
    odd = jax.lax.shift_left(n.astype(jnp.int32), 31)
    r = t - n * jnp.float32(_PI_HI)
    r = r - n * jnp.float32(_PI_LO)
    r2 = r * r
    p = r + r * (r2 * (_S3 + r2 * (_S5 + r2 * (_S7 + r2 * _S9))))
    return (p.view(jnp.int32) ^ odd).view(jnp.float32)


def _act_kernel(x_ref, o_ref):
    x = x_ref[...]
    col = jax.lax.broadcasted_iota(jnp.int32, (1, _N_COLS), 1) % 9

    ax = jnp.abs(x)
    relu = jnp.maximum(x, 0.0)

    earg = jnp.where(col == 6, x * x * -0.5, -ax)
    e = jnp.exp(earg)
    den = 1.0 + e
    rden = 1.0 / den
    sig = jnp.where(x >= 0, 1.0, e) * rden
    sp = relu + jnp.log(den)
    th = jnp.tanh(x)
    s = _sin_poly(x + jnp.where(col == 5, jnp.float32(_HALF_PI), 0.0))

    out = jnp.where(col == 0, x,
          jnp.where(col == 1, relu,
          jnp.where(col == 2, sig,
          jnp.where(col == 3, th,
          jnp.where(col <= 5, s,
          jnp.where(col == 6, e,
          jnp.where(col == 7, ax, sp)))))))
    o_ref[...] = out


@jax.jit
def kernel(x):
    grid = (_N_ROWS // _BLOCK_ROWS,)
    return pl.pallas_call(
        _act_kernel,
        grid=grid,
        in_specs=[pl.BlockSpec((_BLOCK_ROWS, _N_COLS), lambda i: (i, 0))],
        out_specs=pl.BlockSpec((_BLOCK_ROWS, _N_COLS), lambda i: (i, 0)),
        out_shape=jax.ShapeDtypeStruct((_N_ROWS, _N_COLS), jnp.float32),
    )(x)


# speedup vs baseline: 9.3982x; 1.2474x over previous
import jax
import jax.numpy as jnp
from jax.experimental import pallas as pl

_N_ROWS = 131072
_N_COLS = 64
_BLOCK_ROWS = 8192
_HALF = _BLOCK_ROWS // 2

_PI_HI = 3.140625
_PI_LO = 9.67653589793e-4
_INV_PI = 0.3183098861837907
_HALF_PI = 1.5707963267948966

_S3 = -1.6666667163e-01
_S5 = 8.3333337680e-03
_S7 = -1.9841270114e-04
_S9 = 2.7557314297e-06


def _sin_poly(t):
    n = jnp.round(t * jnp.float32(_INV_PI))
    odd = jax.lax.shift_left(n.astype(jnp.int32), 31)
    r = t - n * jnp.float32(_PI_HI)
    r = r - n * jnp.float32(_PI_LO)
    r2 = r * r
    p = r + r * (r2 * (_S3 + r2 * (_S5 + r2 * (_S7 + r2 * _S9))))
    return (p.view(jnp.int32) ^ odd).view(jnp.float32)


def _compute(x, col):
    ax = jnp.abs(x)
    relu = jnp.maximum(x, 0.0)
    earg = jnp.where(col == 6, x * x * -0.5, -ax)
    e = jnp.exp(earg)
    den = 1.0 + e
    rden = 1.0 / den
    sig = jnp.where(x >= 0, 1.0, e) * rden
    sp = relu + jnp.log(den)
    th = jnp.tanh(x)
    s = _sin_poly(x + jnp.where(col == 5, jnp.float32(_HALF_PI), 0.0))
    return jnp.where(col == 0, x,
           jnp.where(col == 1, relu,
           jnp.where(col == 2, sig,
           jnp.where(col == 3, th,
           jnp.where(col <= 5, s,
           jnp.where(col == 6, e,
           jnp.where(col == 7, ax, sp)))))))


def _act_kernel(x_ref, o_ref):
    x = jnp.concatenate([x_ref[:_HALF, :], x_ref[_HALF:, :]], axis=1)
    col = jax.lax.broadcasted_iota(jnp.int32, (1, 128), 1) % 64 % 9
    out = _compute(x, col)
    o_ref[:_HALF, :] = out[:, :64]
    o_ref[_HALF:, :] = out[:, 64:]


@jax.jit
def kernel(x):
    grid = (_N_ROWS // _BLOCK_ROWS,)
    return pl.pallas_call(
        _act_kernel,
        grid=grid,
        in_specs=[pl.BlockSpec((_BLOCK_ROWS, _N_COLS), lambda i: (i, 0))],
        out_specs=pl.BlockSpec((_BLOCK_ROWS, _N_COLS), lambda i: (i, 0)),
        out_shape=jax.ShapeDtypeStruct((_N_ROWS, _N_COLS), jnp.float32),
    )(x)
